# baseline (speedup 1.0000x reference)
import dataclasses

import jax
import jax.numpy as jnp
from jax import lax
from jax.experimental import pallas as pl
from jax.experimental.pallas import tpu as pltpu
from jax.experimental.pallas import tpu_sc as plsc

_D = 100
_R = 32
_DP = 128
_BA = 10
_RB = 2000
_W = 128
_L = 16


def _table_body(c0_ref, c1f_ref, c2m_ref, t_ref):
    ba = c0_ref.shape[0]
    c0 = c0_ref[...].reshape(ba * _R, _R)
    p = jnp.dot(c0, c1f_ref[...], preferred_element_type=jnp.float32)
    p = p.astype(jnp.bfloat16).reshape(ba, _R, _R * _DP)
    pt = p.transpose(0, 2, 1)
    acc = jnp.zeros((ba * _DP, _DP), jnp.float32)
    for o in range(_R // 8):
        lhs = jnp.concatenate(
            [pt[:, (8 * o + m) * _DP:(8 * o + m + 1) * _DP, :] for m in range(8)],
            axis=2,
        ).reshape(ba * _DP, 8 * _R)
        acc = acc + jnp.dot(
            lhs, c2m_ref[8 * _R * o:8 * _R * (o + 1), :],
            preferred_element_type=jnp.float32,
        )
    t_ref[...] = acc


def _build_table(c0, c1f, c2m):
    return pl.pallas_call(
        _table_body,
        grid=(_D // _BA,),
        in_specs=[
            pl.BlockSpec((_BA, _R, _R), lambda g: (g, 0, 0)),
            pl.BlockSpec((_R, _R * _DP), lambda g: (0, 0)),
            pl.BlockSpec((_R * _R, _DP), lambda g: (0, 0)),
        ],
        out_specs=pl.BlockSpec((_BA * _DP, _DP), lambda g: (g, 0)),
        out_shape=jax.ShapeDtypeStruct((_D * _DP, _DP), jnp.float32),
    )(c0, c1f, c2m)


def _gather_table(t2, idx):
    b = idx.shape[0]
    mesh = plsc.VectorSubcoreMesh(core_axis_name="c", subcore_axis_name="s")
    cp = pltpu.CompilerParams()
    if "needs_layout_passes" in pltpu.CompilerParams.__dataclass_fields__:
        cp = dataclasses.replace(cp, needs_layout_passes=False)

    @pl.kernel(
        out_type=jax.ShapeDtypeStruct((b,), jnp.float32),
        mesh=mesh,
        scratch_types=[
            pltpu.VMEM((1, _W), jnp.int32),
            pltpu.VMEM((1, _W), jnp.int32),
            pltpu.VMEM((_W, _DP), jnp.float32),
        ],
        compiler_params=cp,
    )
    def k(t_hbm, idx_hbm, o_hbm, flat_ref, i2_ref, rows_ref):
        def body(idx_v, o_v):
            @pl.loop(0, _W, step=_L)
            def _(c):
                s = (0, pl.ds(c, _L))
                rows16 = lax.iota(jnp.int32, _L) + c
                z = jnp.zeros((_L,), jnp.int32)
                i0c = plsc.load_gather(idx_v, [rows16, z])
                i1c = plsc.load_gather(idx_v, [rows16, z + 1])
                i2c = plsc.load_gather(idx_v, [rows16, z + 2])
                flat_ref.at[*s][...] = i0c * _DP + i1c
                i2_ref.at[*s][...] = i2c
            pltpu.sync_copy(t_hbm.at[flat_ref.at[0]], rows_ref)

            @pl.loop(0, _W, step=_L)
            def _(c):
                rows16 = lax.iota(jnp.int32, _L) + c
                o_v.at[pl.ds(c, _L)][...] = plsc.load_gather(
                    rows_ref, [rows16, i2_ref.at[0, pl.ds(c, _L)][...]]
                )

        pltpu.emit_pipeline(
            body,
            grid=(b // _W,),
            in_specs=[pl.BlockSpec((_W, 3), lambda i: (i, 0))],
            out_specs=[pl.BlockSpec((_W,), lambda i: (i,))],
            core_axis_name=("c", "s"),
            dimension_semantics=(pltpu.PARALLEL,),
        )(idx_hbm, o_hbm)

    return k(t2, idx)


def kernel(index, core0, core1, core2):
    c1f = jnp.pad(
        core1.transpose(1, 2, 0), ((0, 0), (0, 0), (0, _DP - _D))
    ).reshape(_R, _R * _DP)
    c2m = core2.transpose(1, 2, 0).reshape(_R * _R, _D)
    c2m = jnp.pad(c2m, ((0, 0), (0, _DP - _D)))
    t2 = _build_table(
        core0.astype(jnp.bfloat16),
        c1f.astype(jnp.bfloat16),
        c2m.astype(jnp.bfloat16),
    )
    return _gather_table(t2, index.astype(jnp.int32))


# speedup vs baseline: 1.0769x; 1.0769x over previous
import dataclasses

import jax
import jax.numpy as jnp
from jax import lax
from jax.experimental import pallas as pl
from jax.experimental.pallas import tpu as pltpu
from jax.experimental.pallas import tpu_sc as plsc

_D = 100
_R = 32
_DP = 128
_BA = 10
_RB = 2000
_W = 128
_L = 16


def _table_body(c0_ref, c1f_ref, c2m_ref, t_ref):
    ba = c0_ref.shape[0]
    c0 = c0_ref[...].reshape(ba * _R, _R)
    p = jnp.dot(c0, c1f_ref[...], preferred_element_type=jnp.float32)
    p = p.astype(jnp.bfloat16).reshape(ba, _R, _R * _DP)
    pt = p.transpose(0, 2, 1)
    acc = jnp.zeros((ba * _DP, _DP), jnp.float32)
    for o in range(_R // 8):
        lhs = jnp.concatenate(
            [pt[:, (8 * o + m) * _DP:(8 * o + m + 1) * _DP, :] for m in range(8)],
            axis=2,
        ).reshape(ba * _DP, 8 * _R)
        acc = acc + jnp.dot(
            lhs, c2m_ref[8 * _R * o:8 * _R * (o + 1), :],
            preferred_element_type=jnp.float32,
        )
    t_ref[...] = acc


def _build_table(c0, c1f, c2m):
    return pl.pallas_call(
        _table_body,
        grid=(_D // _BA,),
        in_specs=[
            pl.BlockSpec((_BA, _R, _R), lambda g: (g, 0, 0)),
            pl.BlockSpec((_R, _R * _DP), lambda g: (0, 0)),
            pl.BlockSpec((_R * _R, _DP), lambda g: (0, 0)),
        ],
        out_specs=pl.BlockSpec((_BA * _DP, _DP), lambda g: (g, 0)),
        out_shape=jax.ShapeDtypeStruct((_D * _DP, _DP), jnp.float32),
    )(c0, c1f, c2m)


def _gather_table(t2, idx):
    b = idx.shape[0]
    mesh = plsc.VectorSubcoreMesh(core_axis_name="c", subcore_axis_name="s")
    cp = pltpu.CompilerParams()
    if "needs_layout_passes" in pltpu.CompilerParams.__dataclass_fields__:
        cp = dataclasses.replace(cp, needs_layout_passes=False)

    @pl.kernel(
        out_type=jax.ShapeDtypeStruct((b,), jnp.float32),
        mesh=mesh,
        scratch_types=[
            pltpu.VMEM((1, _W), jnp.int32),
            pltpu.VMEM((_W, _DP), jnp.float32),
        ],
        compiler_params=cp,
    )
    def k(t_hbm, i0_hbm, i1_hbm, i2_hbm, o_hbm, flat_ref, rows_ref):
        def body(i0_v, i1_v, i2_v, o_v):
            @pl.loop(0, _W, step=_L)
            def _(c):
                s = (0, pl.ds(c, _L))
                flat_ref.at[*s][...] = i0_v.at[*s][...] * _DP + i1_v.at[*s][...]
            pltpu.sync_copy(t_hbm.at[flat_ref.at[0]], rows_ref)

            @pl.loop(0, _W, step=_L)
            def _(c):
                rows16 = lax.iota(jnp.int32, _L) + c
                o_v.at[pl.ds(c, _L)][...] = plsc.load_gather(
                    rows_ref, [rows16, i2_v.at[0, pl.ds(c, _L)][...]]
                )

        pltpu.emit_pipeline(
            body,
            grid=(b // _W,),
            in_specs=[pl.BlockSpec((1, _W), lambda i: (0, i))] * 3,
            out_specs=[pl.BlockSpec((_W,), lambda i: (i,))],
            core_axis_name=("c", "s"),
            dimension_semantics=(pltpu.PARALLEL,),
        )(i0_hbm, i1_hbm, i2_hbm, o_hbm)

    return k(t2, idx[:, 0].reshape(1, -1), idx[:, 1].reshape(1, -1),
             idx[:, 2].reshape(1, -1))


def kernel(index, core0, core1, core2):
    c1f = jnp.pad(
        core1.transpose(1, 2, 0), ((0, 0), (0, 0), (0, _DP - _D))
    ).reshape(_R, _R * _DP)
    c2m = core2.transpose(1, 2, 0).reshape(_R * _R, _D)
    c2m = jnp.pad(c2m, ((0, 0), (0, _DP - _D)))
    t2 = _build_table(
        core0.astype(jnp.bfloat16),
        c1f.astype(jnp.bfloat16),
        c2m.astype(jnp.bfloat16),
    )
    return _gather_table(t2, index.astype(jnp.int32))


# speedup vs baseline: 1.1051x; 1.0262x over previous
import dataclasses

import jax
import jax.numpy as jnp
from jax import lax
from jax.experimental import pallas as pl
from jax.experimental.pallas import tpu as pltpu
from jax.experimental.pallas import tpu_sc as plsc

_D = 100
_R = 32
_DP = 128
_BA = 20
_W = 256
_L = 16


def _table_body(c0_ref, c1f_ref, c2m_ref, t_ref):
    ba = c0_ref.shape[0]
    c0 = c0_ref[...].astype(jnp.bfloat16).reshape(ba * _R, _R)
    p = jnp.dot(
        c0, c1f_ref[...].astype(jnp.bfloat16), preferred_element_type=jnp.float32
    )
    p = p.astype(jnp.bfloat16).reshape(ba, _R, _R * _DP)
    pt = p.transpose(0, 2, 1)
    acc = jnp.zeros((ba * _DP, _DP), jnp.float32)
    for o in range(_R // 8):
        lhs = jnp.concatenate(
            [pt[:, (8 * o + m) * _DP:(8 * o + m + 1) * _DP, :] for m in range(8)],
            axis=2,
        ).reshape(ba * _DP, 8 * _R)
        acc = acc + jnp.dot(
            lhs, c2m_ref[8 * _R * o:8 * _R * (o + 1), :].astype(jnp.bfloat16),
            preferred_element_type=jnp.float32,
        )
    t_ref[...] = acc


def _build_table(c0, c1f, c2m):
    return pl.pallas_call(
        _table_body,
        grid=(_D // _BA,),
        in_specs=[
            pl.BlockSpec((_BA, _R, _R), lambda g: (g, 0, 0)),
            pl.BlockSpec((_R, _R * _DP), lambda g: (0, 0)),
            pl.BlockSpec((_R * _R, _DP), lambda g: (0, 0)),
        ],
        out_specs=pl.BlockSpec((_BA * _DP, _DP), lambda g: (g, 0)),
        out_shape=jax.ShapeDtypeStruct((_D * _DP, _DP), jnp.float32),
    )(c0, c1f, c2m)


def _gather_table(t2, idx):
    b = idx.shape[0]
    mesh = plsc.VectorSubcoreMesh(core_axis_name="c", subcore_axis_name="s")
    cp = pltpu.CompilerParams()
    if "needs_layout_passes" in pltpu.CompilerParams.__dataclass_fields__:
        cp = dataclasses.replace(cp, needs_layout_passes=False)

    @pl.kernel(
        out_type=jax.ShapeDtypeStruct((b,), jnp.float32),
        mesh=mesh,
        scratch_types=[
            pltpu.VMEM((1, _W), jnp.int32),
            pltpu.VMEM((_W, _DP), jnp.float32),
        ],
        compiler_params=cp,
    )
    def k(t_hbm, i0_hbm, i1_hbm, i2_hbm, o_hbm, flat_ref, rows_ref):
        def body(i0_v, i1_v, i2_v, o_v):
            @pl.loop(0, _W, step=_L)
            def _(c):
                s = (0, pl.ds(c, _L))
                flat_ref.at[*s][...] = i0_v.at[*s][...] * _DP + i1_v.at[*s][...]
            pltpu.sync_copy(t_hbm.at[flat_ref.at[0]], rows_ref)

            @pl.loop(0, _W, step=_L)
            def _(c):
                rows16 = lax.iota(jnp.int32, _L) + c
                o_v.at[pl.ds(c, _L)][...] = plsc.load_gather(
                    rows_ref, [rows16, i2_v.at[0, pl.ds(c, _L)][...]]
                )

        pltpu.emit_pipeline(
            body,
            grid=(b // _W,),
            in_specs=[pl.BlockSpec((1, _W), lambda i: (0, i))] * 3,
            out_specs=[pl.BlockSpec((_W,), lambda i: (i,))],
            core_axis_name=("c", "s"),
            dimension_semantics=(pltpu.PARALLEL,),
        )(i0_hbm, i1_hbm, i2_hbm, o_hbm)

    return k(t2, idx[:, 0].reshape(1, -1), idx[:, 1].reshape(1, -1),
             idx[:, 2].reshape(1, -1))


def kernel(index, core0, core1, core2):
    c1f = jnp.pad(
        core1.transpose(1, 2, 0), ((0, 0), (0, 0), (0, _DP - _D))
    ).reshape(_R, _R * _DP)
    c2m = core2.transpose(1, 2, 0).reshape(_R * _R, _D)
    c2m = jnp.pad(c2m, ((0, 0), (0, _DP - _D)))
    t2 = _build_table(core0, c1f, c2m)
    return _gather_table(t2, index.astype(jnp.int32))


# speedup vs baseline: 1.1138x; 1.0079x over previous
import dataclasses

import jax
import jax.numpy as jnp
from jax import lax
from jax.experimental import pallas as pl
from jax.experimental.pallas import tpu as pltpu
from jax.experimental.pallas import tpu_sc as plsc

_D = 100
_R = 32
_DP = 128
_BA = 20
_W = 512
_L = 16


def _table_body(c0_ref, c1f_ref, c2m_ref, t_ref):
    ba = c0_ref.shape[0]
    c0 = c0_ref[...].astype(jnp.bfloat16).reshape(ba * _R, _R)
    p = jnp.dot(
        c0, c1f_ref[...].astype(jnp.bfloat16), preferred_element_type=jnp.float32
    )
    p = p.astype(jnp.bfloat16).reshape(ba, _R, _R * _DP)
    pt = p.transpose(0, 2, 1)
    acc = jnp.zeros((ba * _DP, _DP), jnp.float32)
    for o in range(_R // 8):
        lhs = jnp.concatenate(
            [pt[:, (8 * o + m) * _DP:(8 * o + m + 1) * _DP, :] for m in range(8)],
            axis=2,
        ).reshape(ba * _DP, 8 * _R)
        acc = acc + jnp.dot(
            lhs, c2m_ref[8 * _R * o:8 * _R * (o + 1), :].astype(jnp.bfloat16),
            preferred_element_type=jnp.float32,
        )
    t_ref[...] = acc


def _build_table(c0, c1f, c2m):
    return pl.pallas_call(
        _table_body,
        grid=(_D // _BA,),
        in_specs=[
            pl.BlockSpec((_BA, _R, _R), lambda g: (g, 0, 0)),
            pl.BlockSpec((_R, _R * _DP), lambda g: (0, 0)),
            pl.BlockSpec((_R * _R, _DP), lambda g: (0, 0)),
        ],
        out_specs=pl.BlockSpec((_BA * _DP, _DP), lambda g: (g, 0)),
        out_shape=jax.ShapeDtypeStruct((_D * _DP, _DP), jnp.float32),
    )(c0, c1f, c2m)


def _gather_table(t2, idx):
    b = idx.shape[0]
    mesh = plsc.VectorSubcoreMesh(core_axis_name="c", subcore_axis_name="s")
    cp = pltpu.CompilerParams()
    if "needs_layout_passes" in pltpu.CompilerParams.__dataclass_fields__:
        cp = dataclasses.replace(cp, needs_layout_passes=False)

    @pl.kernel(
        out_type=jax.ShapeDtypeStruct((b,), jnp.float32),
        mesh=mesh,
        scratch_types=[
            pltpu.VMEM((1, _W), jnp.int32),
            pltpu.VMEM((_W, _DP), jnp.float32),
        ],
        compiler_params=cp,
    )
    def k(t_hbm, i0_hbm, i1_hbm, i2_hbm, o_hbm, flat_ref, rows_ref):
        def body(i0_v, i1_v, i2_v, o_v):
            @pl.loop(0, _W, step=_L)
            def _(c):
                s = (0, pl.ds(c, _L))
                flat_ref.at[*s][...] = i0_v.at[*s][...] * _DP + i1_v.at[*s][...]
            pltpu.sync_copy(t_hbm.at[flat_ref.at[0]], rows_ref)

            @pl.loop(0, _W, step=_L)
            def _(c):
                rows16 = lax.iota(jnp.int32, _L) + c
                o_v.at[pl.ds(c, _L)][...] = plsc.load_gather(
                    rows_ref, [rows16, i2_v.at[0, pl.ds(c, _L)][...]]
                )

        pltpu.emit_pipeline(
            body,
            grid=(b // _W,),
            in_specs=[pl.BlockSpec((1, _W), lambda i: (0, i))] * 3,
            out_specs=[pl.BlockSpec((_W,), lambda i: (i,))],
            core_axis_name=("c", "s"),
            dimension_semantics=(pltpu.PARALLEL,),
        )(i0_hbm, i1_hbm, i2_hbm, o_hbm)

    return k(t2, idx[:, 0].reshape(1, -1), idx[:, 1].reshape(1, -1),
             idx[:, 2].reshape(1, -1))


def kernel(index, core0, core1, core2):
    c1f = jnp.pad(
        core1.transpose(1, 2, 0), ((0, 0), (0, 0), (0, _DP - _D))
    ).reshape(_R, _R * _DP)
    c2m = core2.transpose(1, 2, 0).reshape(_R * _R, _D)
    c2m = jnp.pad(c2m, ((0, 0), (0, _DP - _D)))
    t2 = _build_table(core0, c1f, c2m)
    return _gather_table(t2, index.astype(jnp.int32))
